# baseline (speedup 1.0000x reference)
import functools

import jax
import jax.numpy as jnp
from jax import lax
from jax.experimental import pallas as pl
from jax.experimental.pallas import tpu as pltpu
from jax.experimental.pallas import tpu_sc as plsc

N = 10000
E = 160000
EH = E // 2
H = 128
CHUNK = 128
NCHUNK = EH // CHUNK
NC = 2
NS = 16
NW = NC * NS
ROWS_PER_TILE = 624
BE = 2000
BN = 2000

@functools.cache
def _mesh():
    return plsc.VectorSubcoreMesh(
        core_axis_name="c", subcore_axis_name="s", num_cores=NC, num_subcores=NS)


HW = H // 2
IDXCAP = 40 * CHUNK


@functools.cache
def _gather_kernel():
    return functools.partial(
        pl.kernel,
        out_type=jax.ShapeDtypeStruct((NC, EH, HW), jnp.int32),
        mesh=_mesh(),
        scratch_types=[
            pltpu.VMEM((IDXCAP,), jnp.int32),
            pltpu.VMEM((CHUNK, HW), jnp.int32),
            pltpu.VMEM((CHUNK, HW), jnp.int32),
            pltpu.SemaphoreType.DMA,
            pltpu.SemaphoreType.DMA,
            pltpu.VMEM_SHARED((N, HW), jnp.int32),
        ],
    )(_sc_gather_body)


def _sc_gather(t, j):
    return _gather_kernel()(t, j)


def _sc_gather_body(t_hbm, j_hbm, out_hbm, idx, buf0, buf1,
                    sem0, sem1, table_sh):
    c = lax.axis_index("c")
    s = lax.axis_index("s")
    rbase = s * ROWS_PER_TILE
    tail = N - NS * ROWS_PER_TILE

    pltpu.sync_copy(t_hbm.at[c, pl.ds(rbase, ROWS_PER_TILE)],
                    table_sh.at[pl.ds(rbase, ROWS_PER_TILE)])

    @pl.when(s == 0)
    def _stage_tail():
        pltpu.sync_copy(t_hbm.at[c, pl.ds(NS * ROWS_PER_TILE, tail)],
                        table_sh.at[pl.ds(NS * ROWS_PER_TILE, tail)])

    c_lo = s * NCHUNK // NS
    nj = (s + 1) * NCHUNK // NS - c_lo
    ebase = c_lo * CHUNK
    pltpu.sync_copy(j_hbm.at[c, pl.ds(ebase, IDXCAP)], idx)

    plsc.subcore_barrier()

    def process(j, buf, sem):
        pltpu.sync_copy(table_sh.at[idx.at[pl.ds(j * CHUNK, CHUNK)]], buf)
        pltpu.async_copy(buf, out_hbm.at[c, pl.ds(ebase + j * CHUNK, CHUNK)],
                         sem)

    def wait_write(buf, sem):
        pltpu.make_async_copy(buf, out_hbm.at[c, pl.ds(ebase, CHUNK)],
                              sem).wait()

    def pair_body(k, carry):
        j0 = 2 * k

        @pl.when(j0 >= 2)
        def _w0():
            wait_write(buf0, sem0)

        process(j0, buf0, sem0)

        @pl.when((j0 + 1 < nj) & (j0 >= 2))
        def _w1():
            wait_write(buf1, sem1)

        @pl.when(j0 + 1 < nj)
        def _p1():
            process(j0 + 1, buf1, sem1)

        return carry

    lax.fori_loop(0, (nj + 1) // 2, pair_body, 0)

    wait_write(buf0, sem0)

    @pl.when(nj >= 2)
    def _drain1():
        wait_write(buf1, sem1)


@functools.cache
def _segment_sum_kernel():
    return functools.partial(
        pl.kernel,
        out_type=jax.ShapeDtypeStruct((NC, N, H), jnp.float32),
        mesh=_mesh(),
        scratch_types=[
            pltpu.VMEM((CHUNK,), jnp.int32),
            pltpu.VMEM((CHUNK,), jnp.int32),
            pltpu.VMEM((CHUNK, H), jnp.float32),
            pltpu.VMEM((CHUNK, H), jnp.float32),
            pltpu.SemaphoreType.DMA,
            pltpu.SemaphoreType.DMA,
            pltpu.VMEM_SHARED((N, H), jnp.float32),
        ],
    )(_sc_segment_sum_body)


def _sc_segment_sum(ue, dst, zeros):
    return _segment_sum_kernel()(ue, dst, zeros)


def _sc_segment_sum_body(ue_hbm, dst_hbm, zeros_hbm, out_hbm,
                         idx0, idx1, buf0, buf1, sem0, sem1, agg_sh):
    c = lax.axis_index("c")
    s = lax.axis_index("s")
    rbase = s * ROWS_PER_TILE
    tail = N - NS * ROWS_PER_TILE

    pltpu.sync_copy(zeros_hbm.at[pl.ds(rbase, ROWS_PER_TILE)],
                    agg_sh.at[pl.ds(rbase, ROWS_PER_TILE)])

    @pl.when(s == 0)
    def _zero_tail():
        pltpu.sync_copy(zeros_hbm.at[pl.ds(NS * ROWS_PER_TILE, tail)],
                        agg_sh.at[pl.ds(NS * ROWS_PER_TILE, tail)])

    lo_core = c * NCHUNK // NC
    size = (c + 1) * NCHUNK // NC - lo_core
    c_lo = lo_core + s * size // NS
    nj = lo_core + (s + 1) * size // NS - c_lo
    ebase = c_lo * CHUNK

    plsc.subcore_barrier()

    def start(j, idx, buf, sem):
        base = ebase + j * CHUNK
        pltpu.async_copy(dst_hbm.at[pl.ds(base, CHUNK)], idx, sem)
        pltpu.async_copy(ue_hbm.at[pl.ds(base, CHUNK)], buf, sem)

    def wait(idx, buf, sem):
        pltpu.make_async_copy(dst_hbm.at[pl.ds(ebase, CHUNK)], idx, sem).wait()
        pltpu.make_async_copy(ue_hbm.at[pl.ds(ebase, CHUNK)], buf, sem).wait()

    def process(idx, buf):
        pltpu.sync_copy(buf, agg_sh.at[idx], add=True)

    start(0, idx0, buf0, sem0)

    def pair_body(k, carry):
        j0 = 2 * k

        @pl.when(j0 + 1 < nj)
        def _s1():
            start(j0 + 1, idx1, buf1, sem1)

        wait(idx0, buf0, sem0)
        process(idx0, buf0)

        @pl.when(j0 + 2 < nj)
        def _s0():
            start(j0 + 2, idx0, buf0, sem0)

        @pl.when(j0 + 1 < nj)
        def _p1():
            wait(idx1, buf1, sem1)
            process(idx1, buf1)

        return carry

    lax.fori_loop(0, (nj + 1) // 2, pair_body, 0)
    plsc.subcore_barrier()

    pltpu.sync_copy(agg_sh.at[pl.ds(rbase, ROWS_PER_TILE)],
                    out_hbm.at[c, pl.ds(rbase, ROWS_PER_TILE)])

    @pl.when(s == 0)
    def _out_tail():
        pltpu.sync_copy(agg_sh.at[pl.ds(NS * ROWS_PER_TILE, tail)],
                        out_hbm.at[c, pl.ds(NS * ROWS_PER_TILE, tail)])


def _ln(u, g, b):
    mu = jnp.mean(u, axis=-1, keepdims=True)
    var = jnp.mean((u - mu) * (u - mu), axis=-1, keepdims=True)
    return (u - mu) * lax.rsqrt(var + 1e-5) * g + b


def _mm(a, b):
    return jnp.dot(a, b, preferred_element_type=jnp.float32)


def _enc_edge_body(ea_ref, w1_ref, b1_ref, w2_ref, b2_ref, g_ref, be_ref, out_ref):
    t = jnp.maximum(_mm(ea_ref[...], w1_ref[...]) + b1_ref[...], 0.0)
    u = _mm(t, w2_ref[...]) + b2_ref[...]
    out_ref[...] = _ln(u, g_ref[...], be_ref[...])


def _enc_node_body(x_ref, w1_ref, b1_ref, w2_ref, b2_ref, g_ref, be_ref,
                   wd_ref, ws_ref, h_ref, t_ref):
    t = jnp.maximum(_mm(x_ref[...], w1_ref[...]) + b1_ref[...], 0.0)
    u = _mm(t, w2_ref[...]) + b2_ref[...]
    h = _ln(u, g_ref[...], be_ref[...])
    h_ref[...] = h
    t_ref[0] = _mm(h, wd_ref[...]).astype(jnp.bfloat16)
    t_ref[1] = _mm(h, ws_ref[...]).astype(jnp.bfloat16)


def _edge_layer_body(gte_ref, e_ref, w1e_ref, b1_ref, w2_ref, b2_ref,
                     g_ref, be_ref, out_ref):
    e = e_ref[...]
    wa = gte_ref[0]
    wb = gte_ref[1]
    ev = (lax.bitcast_convert_type(jnp.left_shift(wa, 16), jnp.float32)
          + lax.bitcast_convert_type(jnp.left_shift(wb, 16), jnp.float32))
    od = (lax.bitcast_convert_type(jnp.bitwise_and(wa, jnp.int32(-65536)),
                                   jnp.float32)
          + lax.bitcast_convert_type(jnp.bitwise_and(wb, jnp.int32(-65536)),
                                     jnp.float32))
    gsum = jnp.concatenate([ev, od], axis=-1)
    t = jnp.maximum(gsum + _mm(e, w1e_ref[...]) + b1_ref[...], 0.0)
    u = _mm(t, w2_ref[...]) + b2_ref[...]
    out_ref[...] = _ln(u, g_ref[...], be_ref[...]) + e


def _node_layer_body(h_ref, p_ref, q_ref, v1h_ref, v1a_ref, c1_ref, v2_ref,
                     c2_ref, g_ref, be_ref, wd_ref, ws_ref, h_out, t_out):
    h = h_ref[...]
    agg = (p_ref[0] + p_ref[1]) + (q_ref[0] + q_ref[1])
    t = jnp.maximum(_mm(h, v1h_ref[...]) + _mm(agg, v1a_ref[...]) + c1_ref[...], 0.0)
    u = _mm(t, v2_ref[...]) + c2_ref[...]
    hn = _ln(u, g_ref[...], be_ref[...]) + h
    h_out[...] = hn
    t_out[0] = _mm(hn, wd_ref[...]).astype(jnp.bfloat16)
    t_out[1] = _mm(hn, ws_ref[...]).astype(jnp.bfloat16)


def _node_final_body(h_ref, p_ref, q_ref, v1h_ref, v1a_ref, c1_ref, v2_ref,
                     c2_ref, g_ref, be_ref, dw1_ref, db1_ref, dw2_ref, db2_ref,
                     out_ref):
    h = h_ref[...]
    agg = (p_ref[0] + p_ref[1]) + (q_ref[0] + q_ref[1])
    t = jnp.maximum(_mm(h, v1h_ref[...]) + _mm(agg, v1a_ref[...]) + c1_ref[...], 0.0)
    u = _mm(t, v2_ref[...]) + c2_ref[...]
    hn = _ln(u, g_ref[...], be_ref[...]) + h
    d = jnp.maximum(_mm(hn, dw1_ref[...]) + db1_ref[...], 0.0)
    out_ref[...] = _mm(d, dw2_ref[...]) + db2_ref[...]


def _row_spec(blk, width):
    return pl.BlockSpec((blk, width), lambda i: (i, 0))


def _w(a):
    shape = a.shape
    return pl.BlockSpec(shape, lambda i: tuple(0 for _ in shape))


def _p_spec():
    return pl.BlockSpec((NC, BN, H), lambda i: (0, i, 0))


def _enc_edge(ea, w1, b1, w2, b2, g, be):
    args = (ea, w1, b1, w2, b2, g, be)
    specs = [_row_spec(BE, ea.shape[1])] + [_w(a) for a in args[1:]]
    return pl.pallas_call(
        _enc_edge_body, grid=(EH // BE,), in_specs=specs,
        out_specs=_row_spec(BE, H),
        out_shape=jax.ShapeDtypeStruct((EH, H), jnp.float32))(*args)


def _pair_spec(blk):
    return pl.BlockSpec((NC, blk, H), lambda i: (0, i, 0))


def _enc_node(x, w1, b1, w2, b2, g, be, wd, ws):
    args = (x, w1, b1, w2, b2, g, be, wd, ws)
    specs = [_row_spec(BN, x.shape[1])] + [_w(a) for a in args[1:]]
    return pl.pallas_call(
        _enc_node_body, grid=(N // BN,), in_specs=specs,
        out_specs=(_row_spec(BN, H), _pair_spec(BN)),
        out_shape=(jax.ShapeDtypeStruct((N, H), jnp.float32),
                   jax.ShapeDtypeStruct((NC, N, H), jnp.bfloat16)))(*args)


def _edge_layer(gte, e, w1e, b1, w2, b2, g, be):
    args = (gte, e, w1e, b1, w2, b2, g, be)
    specs = [pl.BlockSpec((NC, BE, HW), lambda i: (0, i, 0)),
             _row_spec(BE, H)] + [_w(a) for a in args[2:]]
    return pl.pallas_call(
        _edge_layer_body, grid=(EH // BE,), in_specs=specs,
        out_specs=_row_spec(BE, H),
        out_shape=jax.ShapeDtypeStruct((EH, H), jnp.float32))(*args)


def _node_layer(h, p, q, v1h, v1a, c1, v2, c2, g, be, wd, ws):
    args = (h, p, q, v1h, v1a, c1, v2, c2, g, be, wd, ws)
    specs = [_row_spec(BN, H), _p_spec(), _p_spec()] + [_w(a) for a in args[3:]]
    return pl.pallas_call(
        _node_layer_body, grid=(N // BN,), in_specs=specs,
        out_specs=(_row_spec(BN, H), _pair_spec(BN)),
        out_shape=(jax.ShapeDtypeStruct((N, H), jnp.float32),
                   jax.ShapeDtypeStruct((NC, N, H), jnp.bfloat16)))(*args)


def _node_final(h, p, q, v1h, v1a, c1, v2, c2, g, be, dw1, db1, dw2, db2):
    args = (h, p, q, v1h, v1a, c1, v2, c2, g, be, dw1, db1, dw2, db2)
    specs = [_row_spec(BN, H), _p_spec(), _p_spec()] + [_w(a) for a in args[3:]]
    return pl.pallas_call(
        _node_final_body, grid=(N // BN,), in_specs=specs,
        out_specs=_row_spec(BN, H),
        out_shape=jax.ShapeDtypeStruct((N, H), jnp.float32))(*args)


def _r(v):
    return v.reshape(1, -1)


def _pack_t(tb):
    return lax.bitcast_convert_type(tb.reshape(NC, N, HW, 2), jnp.int32)


def _perm():
    return jnp.concatenate([jnp.arange(0, H, 2), jnp.arange(1, H, 2)])


def kernel(x, edge_index, edge_attr, mean_vec_x, std_vec_x,
           mean_vec_edge, std_vec_edge, image_3D, params):
    del image_3D
    src = edge_index[0]
    dst = edge_index[1]
    layers = params['layers']

    ne, ee = params['node_enc'], params['edge_enc']
    w1x = ne['W1'] / std_vec_x[:, None]
    b1x = ne['b1'] - (mean_vec_x / std_vec_x) @ ne['W1']
    w1e = ee['W1'] / std_vec_edge[:, None]
    b1e = ee['b1'] - (mean_vec_edge / std_vec_edge) @ ee['W1']

    ew0 = layers[0]['edge']['W1']
    h, T = _enc_node(x, w1x, _r(b1x), ne['W2'], _r(ne['b2']),
                     _r(ne['g']), _r(ne['be']), ew0[:H], ew0[H:2 * H])
    enc_args = (w1e, _r(b1e), ee['W2'], _r(ee['b2']), _r(ee['g']), _r(ee['be']))
    e0 = _enc_edge(edge_attr[:EH], *enc_args)
    e1 = _enc_edge(edge_attr[EH:], *enc_args)

    zeros = jnp.zeros((N, H), jnp.float32)
    dst0, dst1 = dst[:EH], dst[EH:]
    J0 = jnp.stack([dst0, src[:EH]])
    J1 = jnp.stack([dst1, src[EH:]])
    out = None
    for i in range(len(layers)):
        lp = layers[i]
        ewp = lp['edge']
        npr = lp['node']
        P = _perm()
        ew_args = (ewp['W1'][2 * H:][:, P], _r(ewp['b1'][P]), ewp['W2'][P, :],
                   _r(ewp['b2']), _r(ewp['g']), _r(ewp['be']))
        Tp = _pack_t(T)
        g0 = _sc_gather(Tp, J0)
        g1 = _sc_gather(Tp, J1)
        e0 = _edge_layer(g0, e0, *ew_args)
        p0 = _sc_segment_sum(e0, dst0, zeros)
        e1 = _edge_layer(g1, e1, *ew_args)
        p1 = _sc_segment_sum(e1, dst1, zeros)
        nv1 = npr['W1']
        if i + 1 < len(layers):
            ewn = layers[i + 1]['edge']['W1']
            h, T = _node_layer(h, p0, p1, nv1[:H], nv1[H:], _r(npr['b1']),
                               npr['W2'], _r(npr['b2']), _r(npr['g']),
                               _r(npr['be']), ewn[:H], ewn[H:2 * H])
        else:
            dw2 = jnp.zeros((H, H), jnp.float32).at[:, :3].set(params['dec_W2'])
            db2 = jnp.zeros((H,), jnp.float32).at[:3].set(params['dec_b2'])
            out = _node_final(h, p0, p1, nv1[:H], nv1[H:], _r(npr['b1']),
                              npr['W2'], _r(npr['b2']), _r(npr['g']),
                              _r(npr['be']), params['dec_W1'], _r(params['dec_b1']),
                              dw2, _r(db2))
    return out[:, :3]


# speedup vs baseline: 1.1407x; 1.1407x over previous
import functools

import jax
import jax.numpy as jnp
from jax import lax
from jax.experimental import pallas as pl
from jax.experimental.pallas import tpu as pltpu
from jax.experimental.pallas import tpu_sc as plsc

N = 10000
E = 160000
EH = E // 2
H = 128
CHUNK = 128
NCHUNK = EH // CHUNK
NC = 2
NS = 16
NW = NC * NS
ROWS_PER_TILE = 624
BE = 2000
BN = 2000

@functools.cache
def _mesh():
    return plsc.VectorSubcoreMesh(
        core_axis_name="c", subcore_axis_name="s", num_cores=NC, num_subcores=NS)


IDXCAP = 40 * CHUNK


@functools.cache
def _gather_kernel():
    return functools.partial(
        pl.kernel,
        out_type=jax.ShapeDtypeStruct((NC, EH, H), jnp.float32),
        mesh=_mesh(),
        scratch_types=[
            pltpu.VMEM((IDXCAP,), jnp.int32),
            pltpu.VMEM((CHUNK, H), jnp.float32),
            pltpu.VMEM((CHUNK, H), jnp.float32),
            pltpu.SemaphoreType.DMA,
            pltpu.SemaphoreType.DMA,
            pltpu.VMEM_SHARED((N, H), jnp.float32),
        ],
    )(_sc_gather_body)


def _sc_gather(t, j):
    return _gather_kernel()(t, j)


def _sc_gather_body(t_hbm, j_hbm, out_hbm, idx, buf0, buf1,
                    sem0, sem1, table_sh):
    c = lax.axis_index("c")
    s = lax.axis_index("s")
    rbase = s * ROWS_PER_TILE
    tail = N - NS * ROWS_PER_TILE

    pltpu.sync_copy(t_hbm.at[c, pl.ds(rbase, ROWS_PER_TILE)],
                    table_sh.at[pl.ds(rbase, ROWS_PER_TILE)])

    @pl.when(s == 0)
    def _stage_tail():
        pltpu.sync_copy(t_hbm.at[c, pl.ds(NS * ROWS_PER_TILE, tail)],
                        table_sh.at[pl.ds(NS * ROWS_PER_TILE, tail)])

    c_lo = s * NCHUNK // NS
    nj = (s + 1) * NCHUNK // NS - c_lo
    ebase = c_lo * CHUNK
    pltpu.sync_copy(j_hbm.at[c, pl.ds(ebase, IDXCAP)], idx)

    plsc.subcore_barrier()

    def process(j, buf, sem):
        pltpu.sync_copy(table_sh.at[idx.at[pl.ds(j * CHUNK, CHUNK)]], buf)
        pltpu.async_copy(buf, out_hbm.at[c, pl.ds(ebase + j * CHUNK, CHUNK)],
                         sem)

    def wait_write(buf, sem):
        pltpu.make_async_copy(buf, out_hbm.at[c, pl.ds(ebase, CHUNK)],
                              sem).wait()

    def pair_body(k, carry):
        j0 = 2 * k

        @pl.when(j0 >= 2)
        def _w0():
            wait_write(buf0, sem0)

        process(j0, buf0, sem0)

        @pl.when((j0 + 1 < nj) & (j0 >= 2))
        def _w1():
            wait_write(buf1, sem1)

        @pl.when(j0 + 1 < nj)
        def _p1():
            process(j0 + 1, buf1, sem1)

        return carry

    lax.fori_loop(0, (nj + 1) // 2, pair_body, 0)

    wait_write(buf0, sem0)

    @pl.when(nj >= 2)
    def _drain1():
        wait_write(buf1, sem1)


@functools.cache
def _segment_sum_kernel():
    return functools.partial(
        pl.kernel,
        out_type=jax.ShapeDtypeStruct((NC, N, H), jnp.float32),
        mesh=_mesh(),
        scratch_types=[
            pltpu.VMEM((CHUNK,), jnp.int32),
            pltpu.VMEM((CHUNK,), jnp.int32),
            pltpu.VMEM((CHUNK, H), jnp.float32),
            pltpu.VMEM((CHUNK, H), jnp.float32),
            pltpu.SemaphoreType.DMA,
            pltpu.SemaphoreType.DMA,
            pltpu.VMEM_SHARED((N, H), jnp.float32),
        ],
    )(_sc_segment_sum_body)


def _sc_segment_sum(ue, dst, zeros):
    return _segment_sum_kernel()(ue, dst, zeros)


def _sc_segment_sum_body(ue_hbm, dst_hbm, zeros_hbm, out_hbm,
                         idx0, idx1, buf0, buf1, sem0, sem1, agg_sh):
    c = lax.axis_index("c")
    s = lax.axis_index("s")
    rbase = s * ROWS_PER_TILE
    tail = N - NS * ROWS_PER_TILE

    pltpu.sync_copy(zeros_hbm.at[pl.ds(rbase, ROWS_PER_TILE)],
                    agg_sh.at[pl.ds(rbase, ROWS_PER_TILE)])

    @pl.when(s == 0)
    def _zero_tail():
        pltpu.sync_copy(zeros_hbm.at[pl.ds(NS * ROWS_PER_TILE, tail)],
                        agg_sh.at[pl.ds(NS * ROWS_PER_TILE, tail)])

    lo_core = c * NCHUNK // NC
    size = (c + 1) * NCHUNK // NC - lo_core
    c_lo = lo_core + s * size // NS
    nj = lo_core + (s + 1) * size // NS - c_lo
    ebase = c_lo * CHUNK

    plsc.subcore_barrier()

    def start(j, idx, buf, sem):
        base = ebase + j * CHUNK
        pltpu.async_copy(dst_hbm.at[pl.ds(base, CHUNK)], idx, sem)
        pltpu.async_copy(ue_hbm.at[pl.ds(base, CHUNK)], buf, sem)

    def wait(idx, buf, sem):
        pltpu.make_async_copy(dst_hbm.at[pl.ds(ebase, CHUNK)], idx, sem).wait()
        pltpu.make_async_copy(ue_hbm.at[pl.ds(ebase, CHUNK)], buf, sem).wait()

    def process(idx, buf):
        pltpu.sync_copy(buf, agg_sh.at[idx], add=True)

    start(0, idx0, buf0, sem0)

    def pair_body(k, carry):
        j0 = 2 * k

        @pl.when(j0 + 1 < nj)
        def _s1():
            start(j0 + 1, idx1, buf1, sem1)

        wait(idx0, buf0, sem0)
        process(idx0, buf0)

        @pl.when(j0 + 2 < nj)
        def _s0():
            start(j0 + 2, idx0, buf0, sem0)

        @pl.when(j0 + 1 < nj)
        def _p1():
            wait(idx1, buf1, sem1)
            process(idx1, buf1)

        return carry

    lax.fori_loop(0, (nj + 1) // 2, pair_body, 0)
    plsc.subcore_barrier()

    pltpu.sync_copy(agg_sh.at[pl.ds(rbase, ROWS_PER_TILE)],
                    out_hbm.at[c, pl.ds(rbase, ROWS_PER_TILE)])

    @pl.when(s == 0)
    def _out_tail():
        pltpu.sync_copy(agg_sh.at[pl.ds(NS * ROWS_PER_TILE, tail)],
                        out_hbm.at[c, pl.ds(NS * ROWS_PER_TILE, tail)])


def _ln(u, g, b):
    mu = jnp.mean(u, axis=-1, keepdims=True)
    var = jnp.mean((u - mu) * (u - mu), axis=-1, keepdims=True)
    return (u - mu) * lax.rsqrt(var + 1e-5) * g + b


def _mm(a, b):
    return jnp.dot(a, b, preferred_element_type=jnp.float32)


def _enc_edge_body(ea_ref, w1_ref, b1_ref, w2_ref, b2_ref, g_ref, be_ref, out_ref):
    t = jnp.maximum(_mm(ea_ref[...], w1_ref[...]) + b1_ref[...], 0.0)
    u = _mm(t, w2_ref[...]) + b2_ref[...]
    out_ref[...] = _ln(u, g_ref[...], be_ref[...])


def _enc_node_body(x_ref, w1_ref, b1_ref, w2_ref, b2_ref, g_ref, be_ref,
                   wd_ref, ws_ref, h_ref, t_ref):
    t = jnp.maximum(_mm(x_ref[...], w1_ref[...]) + b1_ref[...], 0.0)
    u = _mm(t, w2_ref[...]) + b2_ref[...]
    h = _ln(u, g_ref[...], be_ref[...])
    h_ref[...] = h
    t_ref[0] = _mm(h, wd_ref[...])
    t_ref[1] = _mm(h, ws_ref[...])


def _edge_layer_body(gte_ref, e_ref, w1e_ref, b1_ref, w2_ref, b2_ref,
                     g_ref, be_ref, out_ref):
    e = e_ref[...]
    gsum = gte_ref[0] + gte_ref[1]
    t = jnp.maximum(gsum + _mm(e, w1e_ref[...]) + b1_ref[...], 0.0)
    u = _mm(t, w2_ref[...]) + b2_ref[...]
    out_ref[...] = _ln(u, g_ref[...], be_ref[...]) + e


def _node_layer_body(h_ref, p_ref, q_ref, v1h_ref, v1a_ref, c1_ref, v2_ref,
                     c2_ref, g_ref, be_ref, wd_ref, ws_ref, h_out, t_out):
    h = h_ref[...]
    agg = (p_ref[0] + p_ref[1]) + (q_ref[0] + q_ref[1])
    t = jnp.maximum(_mm(h, v1h_ref[...]) + _mm(agg, v1a_ref[...]) + c1_ref[...], 0.0)
    u = _mm(t, v2_ref[...]) + c2_ref[...]
    hn = _ln(u, g_ref[...], be_ref[...]) + h
    h_out[...] = hn
    t_out[0] = _mm(hn, wd_ref[...])
    t_out[1] = _mm(hn, ws_ref[...])


def _node_final_body(h_ref, p_ref, q_ref, v1h_ref, v1a_ref, c1_ref, v2_ref,
                     c2_ref, g_ref, be_ref, dw1_ref, db1_ref, dw2_ref, db2_ref,
                     out_ref):
    h = h_ref[...]
    agg = (p_ref[0] + p_ref[1]) + (q_ref[0] + q_ref[1])
    t = jnp.maximum(_mm(h, v1h_ref[...]) + _mm(agg, v1a_ref[...]) + c1_ref[...], 0.0)
    u = _mm(t, v2_ref[...]) + c2_ref[...]
    hn = _ln(u, g_ref[...], be_ref[...]) + h
    d = jnp.maximum(_mm(hn, dw1_ref[...]) + db1_ref[...], 0.0)
    out_ref[...] = _mm(d, dw2_ref[...]) + db2_ref[...]


def _row_spec(blk, width):
    return pl.BlockSpec((blk, width), lambda i: (i, 0))


def _w(a):
    shape = a.shape
    return pl.BlockSpec(shape, lambda i: tuple(0 for _ in shape))


def _p_spec():
    return pl.BlockSpec((NC, BN, H), lambda i: (0, i, 0))


def _enc_edge(ea, w1, b1, w2, b2, g, be):
    args = (ea, w1, b1, w2, b2, g, be)
    specs = [_row_spec(BE, ea.shape[1])] + [_w(a) for a in args[1:]]
    return pl.pallas_call(
        _enc_edge_body, grid=(EH // BE,), in_specs=specs,
        out_specs=_row_spec(BE, H),
        out_shape=jax.ShapeDtypeStruct((EH, H), jnp.float32))(*args)


def _pair_spec(blk):
    return pl.BlockSpec((NC, blk, H), lambda i: (0, i, 0))


def _enc_node(x, w1, b1, w2, b2, g, be, wd, ws):
    args = (x, w1, b1, w2, b2, g, be, wd, ws)
    specs = [_row_spec(BN, x.shape[1])] + [_w(a) for a in args[1:]]
    return pl.pallas_call(
        _enc_node_body, grid=(N // BN,), in_specs=specs,
        out_specs=(_row_spec(BN, H), _pair_spec(BN)),
        out_shape=(jax.ShapeDtypeStruct((N, H), jnp.float32),
                   jax.ShapeDtypeStruct((NC, N, H), jnp.float32)))(*args)


def _edge_layer(gte, e, w1e, b1, w2, b2, g, be):
    args = (gte, e, w1e, b1, w2, b2, g, be)
    specs = [_pair_spec(BE), _row_spec(BE, H)] + [_w(a) for a in args[2:]]
    return pl.pallas_call(
        _edge_layer_body, grid=(EH // BE,), in_specs=specs,
        out_specs=_row_spec(BE, H),
        out_shape=jax.ShapeDtypeStruct((EH, H), jnp.float32))(*args)


def _node_layer(h, p, q, v1h, v1a, c1, v2, c2, g, be, wd, ws):
    args = (h, p, q, v1h, v1a, c1, v2, c2, g, be, wd, ws)
    specs = [_row_spec(BN, H), _p_spec(), _p_spec()] + [_w(a) for a in args[3:]]
    return pl.pallas_call(
        _node_layer_body, grid=(N // BN,), in_specs=specs,
        out_specs=(_row_spec(BN, H), _pair_spec(BN)),
        out_shape=(jax.ShapeDtypeStruct((N, H), jnp.float32),
                   jax.ShapeDtypeStruct((NC, N, H), jnp.float32)))(*args)


def _node_final(h, p, q, v1h, v1a, c1, v2, c2, g, be, dw1, db1, dw2, db2):
    args = (h, p, q, v1h, v1a, c1, v2, c2, g, be, dw1, db1, dw2, db2)
    specs = [_row_spec(BN, H), _p_spec(), _p_spec()] + [_w(a) for a in args[3:]]
    return pl.pallas_call(
        _node_final_body, grid=(N // BN,), in_specs=specs,
        out_specs=_row_spec(BN, H),
        out_shape=jax.ShapeDtypeStruct((N, H), jnp.float32))(*args)


def _r(v):
    return v.reshape(1, -1)


def kernel(x, edge_index, edge_attr, mean_vec_x, std_vec_x,
           mean_vec_edge, std_vec_edge, image_3D, params):
    del image_3D
    src = edge_index[0]
    dst = edge_index[1]
    layers = params['layers']

    ne, ee = params['node_enc'], params['edge_enc']
    w1x = ne['W1'] / std_vec_x[:, None]
    b1x = ne['b1'] - (mean_vec_x / std_vec_x) @ ne['W1']
    w1e = ee['W1'] / std_vec_edge[:, None]
    b1e = ee['b1'] - (mean_vec_edge / std_vec_edge) @ ee['W1']

    ew0 = layers[0]['edge']['W1']
    h, T = _enc_node(x, w1x, _r(b1x), ne['W2'], _r(ne['b2']),
                     _r(ne['g']), _r(ne['be']), ew0[:H], ew0[H:2 * H])
    enc_args = (w1e, _r(b1e), ee['W2'], _r(ee['b2']), _r(ee['g']), _r(ee['be']))
    e0 = _enc_edge(edge_attr[:EH], *enc_args)
    e1 = _enc_edge(edge_attr[EH:], *enc_args)

    zeros = jnp.zeros((N, H), jnp.float32)
    dst0, dst1 = dst[:EH], dst[EH:]
    J0 = jnp.stack([dst0, src[:EH]])
    J1 = jnp.stack([dst1, src[EH:]])
    out = None
    for i in range(len(layers)):
        lp = layers[i]
        ewp = lp['edge']
        npr = lp['node']
        ew_args = (ewp['W1'][2 * H:], _r(ewp['b1']), ewp['W2'],
                   _r(ewp['b2']), _r(ewp['g']), _r(ewp['be']))
        g0 = _sc_gather(T, J0)
        g1 = _sc_gather(T, J1)
        e0 = _edge_layer(g0, e0, *ew_args)
        p0 = _sc_segment_sum(e0, dst0, zeros)
        e1 = _edge_layer(g1, e1, *ew_args)
        p1 = _sc_segment_sum(e1, dst1, zeros)
        nv1 = npr['W1']
        if i + 1 < len(layers):
            ewn = layers[i + 1]['edge']['W1']
            h, T = _node_layer(h, p0, p1, nv1[:H], nv1[H:], _r(npr['b1']),
                               npr['W2'], _r(npr['b2']), _r(npr['g']),
                               _r(npr['be']), ewn[:H], ewn[H:2 * H])
        else:
            dw2 = jnp.zeros((H, H), jnp.float32).at[:, :3].set(params['dec_W2'])
            db2 = jnp.zeros((H,), jnp.float32).at[:3].set(params['dec_b2'])
            out = _node_final(h, p0, p1, nv1[:H], nv1[H:], _r(npr['b1']),
                              npr['W2'], _r(npr['b2']), _r(npr['g']),
                              _r(npr['be']), params['dec_W1'], _r(params['dec_b1']),
                              dw2, _r(db2))
    return out[:, :3]
